# baseline (speedup 1.0000x reference)
import jax
import jax.numpy as jnp
from jax.experimental import pallas as pl
from jax.experimental.pallas import tpu as pltpu


_CHUNK_SIZES = (1024, 3072, 4096)
_CHUNK_OFFS = (0, 1024, 4096)


def _copy_body(src_hbm, dst_hbm, buf, in_sems, out_sems):
    n = len(_CHUNK_SIZES)

    def in_copy(i):
        return pltpu.make_async_copy(
            src_hbm.at[pl.ds(_CHUNK_OFFS[i], _CHUNK_SIZES[i])],
            buf.at[pl.ds(_CHUNK_OFFS[i], _CHUNK_SIZES[i])], in_sems.at[i])

    def out_copy(i):
        return pltpu.make_async_copy(
            buf.at[pl.ds(_CHUNK_OFFS[i], _CHUNK_SIZES[i])],
            dst_hbm.at[0, pl.ds(_CHUNK_OFFS[i], _CHUNK_SIZES[i])],
            out_sems.at[i])

    for i in range(n):
        in_copy(i).start()
    for i in range(n):
        in_copy(i).wait()
        out_copy(i).start()
    for i in range(n):
        out_copy(i).wait()


def kernel(x, pe_weight):
    seq_len = x.shape[1]
    hidden = pe_weight.shape[1]
    n = len(_CHUNK_SIZES)
    return pl.pallas_call(
        _copy_body,
        out_shape=jax.ShapeDtypeStruct((1, seq_len, hidden), pe_weight.dtype),
        in_specs=[pl.BlockSpec(memory_space=pl.ANY)],
        out_specs=pl.BlockSpec(memory_space=pl.ANY),
        scratch_shapes=[
            pltpu.VMEM((seq_len, hidden), pe_weight.dtype),
            pltpu.SemaphoreType.DMA((n,)),
            pltpu.SemaphoreType.DMA((n,)),
        ],
    )(pe_weight)
